# Initial kernel scaffold; baseline (speedup 1.0000x reference)
import jax
import jax.numpy as jnp
from jax import lax
from jax.experimental import pallas as pl
from jax.experimental.pallas import tpu as pltpu
from jax.experimental.pallas import tpu_sc as plsc

N_NODES = 10000
N_EDGES = 320000
D_IN = 128
D_HID = 16
D_OUT = 128

NTILES = 16
NPT = N_NODES // NTILES
CH = 100
NCHUNKS_TOTAL = N_EDGES // CH
NCH = NCHUNKS_TOTAL // NTILES
EPS = 1e-5


ROWS_BLK = 1000
GRID_ROWS = N_NODES // ROWS_BLK


def _gelu(x):
    return 0.5 * x * (1.0 + lax.erf(x * (2.0 ** -0.5)))


def _tc1_body(x_ref, w1_ref, b1_ref, g1_ref, be1_ref, h_ref):
    x = x_ref[...]
    h = lax.dot_general(x, w1_ref[...], (((1,), (1,)), ((), ())),
                        preferred_element_type=jnp.float32)
    h = _gelu(h + b1_ref[...])
    mu = jnp.mean(h, axis=-1, keepdims=True)
    var = jnp.mean((h - mu) ** 2, axis=-1, keepdims=True)
    h_ref[...] = (h - mu) / jnp.sqrt(var + EPS) * g1_ref[...] + be1_ref[...]


def _tc1(x, W1, b1, g1, be1):
    return pl.pallas_call(
        _tc1_body,
        out_shape=jax.ShapeDtypeStruct((N_NODES, D_HID), jnp.float32),
        grid=(GRID_ROWS,),
        in_specs=[
            pl.BlockSpec((ROWS_BLK, D_IN), lambda i: (i, 0)),
            pl.BlockSpec((D_HID, D_IN), lambda i: (0, 0)),
            pl.BlockSpec((1, D_HID), lambda i: (0, 0)),
            pl.BlockSpec((1, D_HID), lambda i: (0, 0)),
            pl.BlockSpec((1, D_HID), lambda i: (0, 0)),
        ],
        out_specs=pl.BlockSpec((ROWS_BLK, D_HID), lambda i: (i, 0)),
    )(x, W1, b1.reshape(1, D_HID), g1.reshape(1, D_HID), be1.reshape(1, D_HID))


def _rsqrt16(x):
    i = plsc.bitcast(x, jnp.int32)
    i = jnp.int32(0x5F3759DF) - (i >> 1)
    y = plsc.bitcast(i, jnp.float32)
    for _ in range(3):
        y = y * (1.5 - 0.5 * x * y * y)
    return y


def _sc_body(rows_hbm, cols_hbm, h_hbm, c6_hbm, c8_hbm, c10_hbm,
             u_sh, s_sh, rowix, colix, gbuf, onesb, nbufS, nbufU, d2b, sdb,
             zbuf):
    cid = lax.axis_index("c")
    tid = lax.axis_index("s")

    @pl.when(cid == 0)
    def _work():
        nbase = tid * NPT
        cbase = tid * NCH

        pltpu.sync_copy(rows_hbm.at[pl.ds(cbase, NCH), :], rowix)
        pltpu.sync_copy(cols_hbm.at[pl.ds(cbase, NCH), :], colix)

        def _fill_const(i, c):
            zbuf[i, :] = jnp.zeros((D_HID,), jnp.float32)
            return c
        lax.fori_loop(0, NPT, _fill_const, 0)

        def _fill_ones(i, c):
            onesb[i, :] = jnp.ones((D_HID,), jnp.float32)
            return c
        lax.fori_loop(0, CH, _fill_ones, 0)

        pltpu.sync_copy(zbuf, s_sh.at[pl.ds(nbase, NPT), :])
        plsc.subcore_barrier()

        def _hist(j, c):
            pltpu.sync_copy(onesb, s_sh.at[colix.at[j]], add=True)
            return c
        lax.fori_loop(0, NCH, _hist, 0)
        plsc.subcore_barrier()

        pltpu.sync_copy(s_sh.at[pl.ds(nbase, NPT), :], nbufS)
        pltpu.sync_copy(h_hbm.at[pl.ds(nbase, NPT), :], nbufU)

        def _setup(i, c):
            deg = nbufS[i, :] + 1.0
            r = _rsqrt16(deg)
            d2b[i, :] = 1.0 / deg
            sdb[i, :] = deg * r
            nbufU[i, :] = r * nbufU[i, :]
            return c
        lax.fori_loop(0, NPT, _setup, 0)

        pltpu.sync_copy(nbufU, u_sh.at[pl.ds(nbase, NPT), :])
        pltpu.sync_copy(zbuf, s_sh.at[pl.ds(nbase, NPT), :])
        plsc.subcore_barrier()

        taps = {6: c6_hbm, 8: c8_hbm, 10: c10_hbm}
        for p in range(1, 11):
            def _edges(j, c):
                pltpu.sync_copy(u_sh.at[rowix.at[j]], gbuf)
                pltpu.sync_copy(gbuf, s_sh.at[colix.at[j]], add=True)
                return c
            lax.fori_loop(0, NCH, _edges, 0)
            plsc.subcore_barrier()

            pltpu.sync_copy(s_sh.at[pl.ds(nbase, NPT), :], nbufS)

            def _update(i, c):
                nbufU[i, :] = d2b[i, :] * (nbufS[i, :] + nbufU[i, :])
                return c
            lax.fori_loop(0, NPT, _update, 0)

            pltpu.sync_copy(nbufU, u_sh.at[pl.ds(nbase, NPT), :])
            pltpu.sync_copy(zbuf, s_sh.at[pl.ds(nbase, NPT), :])

            if p in taps:
                def _tap(i, c):
                    nbufS[i, :] = sdb[i, :] * nbufU[i, :]
                    return c
                lax.fori_loop(0, NPT, _tap, 0)
                pltpu.sync_copy(nbufS, taps[p].at[pl.ds(nbase, NPT), :])
            plsc.subcore_barrier()


def _sc_prop(rows_r, cols_r, h):
    mesh = plsc.VectorSubcoreMesh(core_axis_name="c", subcore_axis_name="s")
    f = pl.kernel(
        _sc_body,
        out_type=(
            jax.ShapeDtypeStruct((N_NODES, D_HID), jnp.float32),
            jax.ShapeDtypeStruct((N_NODES, D_HID), jnp.float32),
            jax.ShapeDtypeStruct((N_NODES, D_HID), jnp.float32),
        ),
        mesh=mesh,
        scratch_types=[
            pltpu.VMEM_SHARED((N_NODES, D_HID), jnp.float32),
            pltpu.VMEM_SHARED((N_NODES, D_HID), jnp.float32),
            pltpu.VMEM((NCH, CH), jnp.int32),
            pltpu.VMEM((NCH, CH), jnp.int32),
            pltpu.VMEM((CH, D_HID), jnp.float32),
            pltpu.VMEM((CH, D_HID), jnp.float32),
            pltpu.VMEM((NPT, D_HID), jnp.float32),
            pltpu.VMEM((NPT, D_HID), jnp.float32),
            pltpu.VMEM((NPT, D_HID), jnp.float32),
            pltpu.VMEM((NPT, D_HID), jnp.float32),
            pltpu.VMEM((NPT, D_HID), jnp.float32),
        ],
    )
    return f(rows_r, cols_r, h)


D_CAT = 3 * D_HID


def _tc2_body(c6_ref, c8_ref, c10_ref, w6_ref, b6_ref, w8_ref, b8_ref,
              w10_ref, b10_ref, g2_ref, be2_ref, w2_ref, b2_ref, out_ref):
    def lin(c_ref, w_ref, b_ref):
        return lax.dot_general(c_ref[...], w_ref[...], (((1,), (1,)), ((), ())),
                               preferred_element_type=jnp.float32) + b_ref[...]

    t6 = _gelu(lin(c6_ref, w6_ref, b6_ref))
    t8 = _gelu(lin(c8_ref, w8_ref, b8_ref))
    t10 = _gelu(lin(c10_ref, w10_ref, b10_ref))

    s = jnp.sum(t6, axis=-1, keepdims=True) + jnp.sum(t8, axis=-1, keepdims=True) \
        + jnp.sum(t10, axis=-1, keepdims=True)
    mu = s / D_CAT
    v = (jnp.sum((t6 - mu) ** 2, axis=-1, keepdims=True)
         + jnp.sum((t8 - mu) ** 2, axis=-1, keepdims=True)
         + jnp.sum((t10 - mu) ** 2, axis=-1, keepdims=True)) / D_CAT
    inv = 1.0 / jnp.sqrt(v + EPS)

    g2 = g2_ref[...]
    be2 = be2_ref[...]
    w2 = w2_ref[...]
    acc = jnp.zeros_like(out_ref[...]) + b2_ref[...]
    for k, t in enumerate((t6, t8, t10)):
        nk = (t - mu) * inv * g2[:, k * D_HID:(k + 1) * D_HID] \
            + be2[:, k * D_HID:(k + 1) * D_HID]
        acc = acc + lax.dot_general(
            nk, w2[:, k * D_HID:(k + 1) * D_HID], (((1,), (1,)), ((), ())),
            preferred_element_type=jnp.float32)
    out_ref[...] = acc


def _tc2(c6, c8, c10, W6, b6, W8, b8, W10, b10, g2, be2, W2, b2):
    blk16 = pl.BlockSpec((ROWS_BLK, D_HID), lambda i: (i, 0))
    w16 = pl.BlockSpec((D_HID, D_HID), lambda i: (0, 0))
    v16 = pl.BlockSpec((1, D_HID), lambda i: (0, 0))
    v48 = pl.BlockSpec((1, D_CAT), lambda i: (0, 0))
    return pl.pallas_call(
        _tc2_body,
        out_shape=jax.ShapeDtypeStruct((N_NODES, D_OUT), jnp.float32),
        grid=(GRID_ROWS,),
        in_specs=[
            blk16, blk16, blk16,
            w16, v16, w16, v16, w16, v16,
            v48, v48,
            pl.BlockSpec((D_OUT, D_CAT), lambda i: (0, 0)),
            pl.BlockSpec((1, D_OUT), lambda i: (0, 0)),
        ],
        out_specs=pl.BlockSpec((ROWS_BLK, D_OUT), lambda i: (i, 0)),
    )(c6, c8, c10,
      W6, b6.reshape(1, D_HID), W8, b8.reshape(1, D_HID),
      W10, b10.reshape(1, D_HID),
      g2.reshape(1, D_CAT), be2.reshape(1, D_CAT),
      W2, b2.reshape(1, D_OUT))


def kernel(x, edge_index, W1, b1, W6, b6, W8, b8, W10, b10,
           g1, be1, g2, be2, W2, b2):
    h = _tc1(x, W1, b1, g1, be1)
    rows_r = edge_index[0].reshape(NCHUNKS_TOTAL, CH)
    cols_r = edge_index[1].reshape(NCHUNKS_TOTAL, CH)
    c6, c8, c10 = _sc_prop(rows_r, cols_r, h)
    return _tc2(c6, c8, c10, W6, b6, W8, b8, W10, b10, g2, be2, W2, b2)


# speedup vs baseline: 32.3080x; 32.3080x over previous
import jax
import jax.numpy as jnp
from jax import lax
from jax.experimental import pallas as pl
from jax.experimental.pallas import tpu as pltpu
from jax.experimental.pallas import tpu_sc as plsc

N_NODES = 10000
N_EDGES = 320000
D_IN = 128
D_HID = 16
D_OUT = 128

NTILES = 16
NPT = N_NODES // NTILES
CH = 100
NCHUNKS_TOTAL = N_EDGES // CH
NCH = NCHUNKS_TOTAL // NTILES
EPS = 1e-5


ROWS_BLK = 1000
GRID_ROWS = N_NODES // ROWS_BLK


def _gelu(x):
    return 0.5 * x * (1.0 + lax.erf(x * (2.0 ** -0.5)))


def _tc1_body(x_ref, w1_ref, b1_ref, g1_ref, be1_ref, h_ref):
    x = x_ref[...]
    h = lax.dot_general(x, w1_ref[...], (((1,), (1,)), ((), ())),
                        preferred_element_type=jnp.float32)
    h = _gelu(h + b1_ref[...])
    mu = jnp.mean(h, axis=-1, keepdims=True)
    var = jnp.mean((h - mu) ** 2, axis=-1, keepdims=True)
    h_ref[...] = (h - mu) / jnp.sqrt(var + EPS) * g1_ref[...] + be1_ref[...]


def _tc1(x, W1, b1, g1, be1):
    return pl.pallas_call(
        _tc1_body,
        out_shape=jax.ShapeDtypeStruct((N_NODES, D_HID), jnp.float32),
        grid=(GRID_ROWS,),
        in_specs=[
            pl.BlockSpec((ROWS_BLK, D_IN), lambda i: (i, 0)),
            pl.BlockSpec((D_HID, D_IN), lambda i: (0, 0)),
            pl.BlockSpec((1, D_HID), lambda i: (0, 0)),
            pl.BlockSpec((1, D_HID), lambda i: (0, 0)),
            pl.BlockSpec((1, D_HID), lambda i: (0, 0)),
        ],
        out_specs=pl.BlockSpec((ROWS_BLK, D_HID), lambda i: (i, 0)),
    )(x, W1, b1.reshape(1, D_HID), g1.reshape(1, D_HID), be1.reshape(1, D_HID))


def _rsqrt16(x):
    i = plsc.bitcast(x, jnp.int32)
    i = jnp.int32(0x5F3759DF) - (i >> 1)
    y = plsc.bitcast(i, jnp.float32)
    for _ in range(3):
        y = y * (1.5 - 0.5 * x * y * y)
    return y


def _sc_body(rows_hbm, cols_hbm, h_hbm, c6_hbm, c8_hbm, c10_hbm,
             u_sh, s_sh, rowix, colix, gbuf, onesb, nbufS, nbufU, d2b, sdb,
             zbuf):
    cid = lax.axis_index("c")
    tid = lax.axis_index("s")

    @pl.when(cid == 0)
    def _work():
        nbase = tid * NPT
        cbase = tid * NCH

        pltpu.sync_copy(rows_hbm.at[pl.ds(cbase, NCH), :], rowix)
        pltpu.sync_copy(cols_hbm.at[pl.ds(cbase, NCH), :], colix)

        def _fill_const(i, c):
            zbuf[i, :] = jnp.zeros((D_HID,), jnp.float32)
            return c
        lax.fori_loop(0, NPT, _fill_const, 0)

        def _fill_ones(i, c):
            onesb[i, :] = jnp.ones((D_HID,), jnp.float32)
            return c
        lax.fori_loop(0, CH, _fill_ones, 0)

        pltpu.sync_copy(zbuf, s_sh.at[pl.ds(nbase, NPT), :])
        plsc.subcore_barrier()

        def _hist(j, c):
            pltpu.sync_copy(onesb, s_sh.at[colix.at[j]], add=True)
            return c
        lax.fori_loop(0, NCH, _hist, 0)
        plsc.subcore_barrier()

        pltpu.sync_copy(s_sh.at[pl.ds(nbase, NPT), :], nbufS)
        pltpu.sync_copy(h_hbm.at[pl.ds(nbase, NPT), :], nbufU)

        def _setup(i, c):
            deg = nbufS[i, :] + 1.0
            r = _rsqrt16(deg)
            d2b[i, :] = 1.0 / deg
            sdb[i, :] = deg * r
            nbufU[i, :] = r * nbufU[i, :]
            return c
        lax.fori_loop(0, NPT, _setup, 0)

        pltpu.sync_copy(nbufU, u_sh.at[pl.ds(nbase, NPT), :])
        pltpu.sync_copy(zbuf, s_sh.at[pl.ds(nbase, NPT), :])
        plsc.subcore_barrier()

        taps = {6: c6_hbm, 8: c8_hbm, 10: c10_hbm}
        for p in range(1, 11):
            def _edges(j, c):
                pltpu.sync_copy(u_sh.at[rowix.at[j]], gbuf)
                pltpu.sync_copy(gbuf, s_sh.at[colix.at[j]], add=True)
                return c
            lax.fori_loop(0, NCH, _edges, 0)
            plsc.subcore_barrier()

            pltpu.sync_copy(s_sh.at[pl.ds(nbase, NPT), :], nbufS)

            def _update(i, c):
                nbufU[i, :] = d2b[i, :] * (nbufS[i, :] + nbufU[i, :])
                return c
            lax.fori_loop(0, NPT, _update, 0)

            pltpu.sync_copy(nbufU, u_sh.at[pl.ds(nbase, NPT), :])
            pltpu.sync_copy(zbuf, s_sh.at[pl.ds(nbase, NPT), :])

            if p in taps:
                def _tap(i, c):
                    nbufS[i, :] = sdb[i, :] * nbufU[i, :]
                    return c
                lax.fori_loop(0, NPT, _tap, 0)
                pltpu.sync_copy(nbufS, taps[p].at[pl.ds(nbase, NPT), :])
            plsc.subcore_barrier()


def _sc_prop(rows_r, cols_r, h):
    mesh = plsc.VectorSubcoreMesh(core_axis_name="c", subcore_axis_name="s")
    f = pl.kernel(
        _sc_body,
        out_type=(
            jax.ShapeDtypeStruct((N_NODES, D_HID), jnp.float32),
            jax.ShapeDtypeStruct((N_NODES, D_HID), jnp.float32),
            jax.ShapeDtypeStruct((N_NODES, D_HID), jnp.float32),
        ),
        mesh=mesh,
        compiler_params=pltpu.CompilerParams(use_tc_tiling_on_sc=False,
                                              needs_layout_passes=False),
        scratch_types=[
            pltpu.VMEM_SHARED((N_NODES, D_HID), jnp.float32),
            pltpu.VMEM_SHARED((N_NODES, D_HID), jnp.float32),
            pltpu.VMEM((NCH, CH), jnp.int32),
            pltpu.VMEM((NCH, CH), jnp.int32),
            pltpu.VMEM((CH, D_HID), jnp.float32),
            pltpu.VMEM((CH, D_HID), jnp.float32),
            pltpu.VMEM((NPT, D_HID), jnp.float32),
            pltpu.VMEM((NPT, D_HID), jnp.float32),
            pltpu.VMEM((NPT, D_HID), jnp.float32),
            pltpu.VMEM((NPT, D_HID), jnp.float32),
            pltpu.VMEM((NPT, D_HID), jnp.float32),
        ],
    )
    return f(rows_r, cols_r, h)


D_CAT = 3 * D_HID


def _tc2_body(c6_ref, c8_ref, c10_ref, w6_ref, b6_ref, w8_ref, b8_ref,
              w10_ref, b10_ref, g2_ref, be2_ref, w2_ref, b2_ref, out_ref):
    def lin(c_ref, w_ref, b_ref):
        return lax.dot_general(c_ref[...], w_ref[...], (((1,), (1,)), ((), ())),
                               preferred_element_type=jnp.float32) + b_ref[...]

    t6 = _gelu(lin(c6_ref, w6_ref, b6_ref))
    t8 = _gelu(lin(c8_ref, w8_ref, b8_ref))
    t10 = _gelu(lin(c10_ref, w10_ref, b10_ref))

    s = jnp.sum(t6, axis=-1, keepdims=True) + jnp.sum(t8, axis=-1, keepdims=True) \
        + jnp.sum(t10, axis=-1, keepdims=True)
    mu = s / D_CAT
    v = (jnp.sum((t6 - mu) ** 2, axis=-1, keepdims=True)
         + jnp.sum((t8 - mu) ** 2, axis=-1, keepdims=True)
         + jnp.sum((t10 - mu) ** 2, axis=-1, keepdims=True)) / D_CAT
    inv = 1.0 / jnp.sqrt(v + EPS)

    g2 = g2_ref[...]
    be2 = be2_ref[...]
    w2 = w2_ref[...]
    acc = jnp.zeros_like(out_ref[...]) + b2_ref[...]
    for k, t in enumerate((t6, t8, t10)):
        nk = (t - mu) * inv * g2[:, k * D_HID:(k + 1) * D_HID] \
            + be2[:, k * D_HID:(k + 1) * D_HID]
        acc = acc + lax.dot_general(
            nk, w2[:, k * D_HID:(k + 1) * D_HID], (((1,), (1,)), ((), ())),
            preferred_element_type=jnp.float32)
    out_ref[...] = acc


def _tc2(c6, c8, c10, W6, b6, W8, b8, W10, b10, g2, be2, W2, b2):
    blk16 = pl.BlockSpec((ROWS_BLK, D_HID), lambda i: (i, 0))
    w16 = pl.BlockSpec((D_HID, D_HID), lambda i: (0, 0))
    v16 = pl.BlockSpec((1, D_HID), lambda i: (0, 0))
    v48 = pl.BlockSpec((1, D_CAT), lambda i: (0, 0))
    return pl.pallas_call(
        _tc2_body,
        out_shape=jax.ShapeDtypeStruct((N_NODES, D_OUT), jnp.float32),
        grid=(GRID_ROWS,),
        in_specs=[
            blk16, blk16, blk16,
            w16, v16, w16, v16, w16, v16,
            v48, v48,
            pl.BlockSpec((D_OUT, D_CAT), lambda i: (0, 0)),
            pl.BlockSpec((1, D_OUT), lambda i: (0, 0)),
        ],
        out_specs=pl.BlockSpec((ROWS_BLK, D_OUT), lambda i: (i, 0)),
    )(c6, c8, c10,
      W6, b6.reshape(1, D_HID), W8, b8.reshape(1, D_HID),
      W10, b10.reshape(1, D_HID),
      g2.reshape(1, D_CAT), be2.reshape(1, D_CAT),
      W2, b2.reshape(1, D_OUT))


def kernel(x, edge_index, W1, b1, W6, b6, W8, b8, W10, b10,
           g1, be1, g2, be2, W2, b2):
    h = _tc1(x, W1, b1, g1, be1)
    rows_r = edge_index[0].reshape(NCHUNKS_TOTAL, CH)
    cols_r = edge_index[1].reshape(NCHUNKS_TOTAL, CH)
    c6, c8, c10 = _sc_prop(rows_r, cols_r, h)
    return _tc2(c6, c8, c10, W6, b6, W8, b8, W10, b10, g2, be2, W2, b2)


# speedup vs baseline: 49.5538x; 1.5338x over previous
import jax
import jax.numpy as jnp
from jax import lax
from jax.experimental import pallas as pl
from jax.experimental.pallas import tpu as pltpu
from jax.experimental.pallas import tpu_sc as plsc

N_NODES = 10000
N_EDGES = 320000
D_IN = 128
D_HID = 16
D_OUT = 128

NTILES = 16
NPT = N_NODES // NTILES
CH = 125
NCHUNKS_TOTAL = N_EDGES // CH
NCH = NCHUNKS_TOTAL // NTILES
EPS = 1e-5


ROWS_BLK = 1000
GRID_ROWS = N_NODES // ROWS_BLK


def _gelu(x):
    return 0.5 * x * (1.0 + lax.erf(x * (2.0 ** -0.5)))


def _tc1_body(x_ref, w1_ref, b1_ref, g1_ref, be1_ref, h_ref):
    x = x_ref[...]
    h = lax.dot_general(x, w1_ref[...], (((1,), (1,)), ((), ())),
                        preferred_element_type=jnp.float32)
    h = _gelu(h + b1_ref[...])
    mu = jnp.mean(h, axis=-1, keepdims=True)
    var = jnp.mean((h - mu) ** 2, axis=-1, keepdims=True)
    h_ref[...] = (h - mu) / jnp.sqrt(var + EPS) * g1_ref[...] + be1_ref[...]


def _tc1(x, W1, b1, g1, be1):
    return pl.pallas_call(
        _tc1_body,
        out_shape=jax.ShapeDtypeStruct((N_NODES, D_HID), jnp.float32),
        grid=(GRID_ROWS,),
        in_specs=[
            pl.BlockSpec((ROWS_BLK, D_IN), lambda i: (i, 0)),
            pl.BlockSpec((D_HID, D_IN), lambda i: (0, 0)),
            pl.BlockSpec((1, D_HID), lambda i: (0, 0)),
            pl.BlockSpec((1, D_HID), lambda i: (0, 0)),
            pl.BlockSpec((1, D_HID), lambda i: (0, 0)),
        ],
        out_specs=pl.BlockSpec((ROWS_BLK, D_HID), lambda i: (i, 0)),
    )(x, W1, b1.reshape(1, D_HID), g1.reshape(1, D_HID), be1.reshape(1, D_HID))


def _rsqrt16(x):
    i = plsc.bitcast(x, jnp.int32)
    i = jnp.int32(0x5F3759DF) - (i >> 1)
    y = plsc.bitcast(i, jnp.float32)
    for _ in range(3):
        y = y * (1.5 - 0.5 * x * y * y)
    return y


def _sc_body(rows_hbm, cols_hbm, h_hbm, c6_hbm, c8_hbm, c10_hbm,
             u_sh, s_sh, rowix, colix, gbuf0, gbuf1, onesb, nbufS, nbufU,
             d2b, sdb, zbuf, gsem0, gsem1, ssem0, ssem1):
    cid = lax.axis_index("c")
    tid = lax.axis_index("s")

    @pl.when(cid == 0)
    def _work():
        nbase = tid * NPT
        cbase = tid * NCH

        pltpu.sync_copy(rows_hbm.at[pl.ds(cbase, NCH), :], rowix)
        pltpu.sync_copy(cols_hbm.at[pl.ds(cbase, NCH), :], colix)

        gbufs = (gbuf0, gbuf1)
        gsems = (gsem0, gsem1)
        ssems = (ssem0, ssem1)

        def g_fire(j, b):
            pltpu.async_copy(u_sh.at[rowix.at[j]], gbufs[b], gsems[b])

        def g_wait(j, b):
            pltpu.make_async_copy(u_sh.at[rowix.at[j]], gbufs[b],
                                  gsems[b]).wait()

        def s_fire(j, b, src=None):
            pltpu.async_copy(src if src is not None else gbufs[b],
                             s_sh.at[colix.at[j]], ssems[b], add=True)

        def s_wait(j, b, src=None):
            pltpu.make_async_copy(src if src is not None else gbufs[b],
                                  s_sh.at[colix.at[j]], ssems[b]).wait()

        def _fill_const(i, c):
            zbuf[i, :] = jnp.zeros((D_HID,), jnp.float32)
            return c
        lax.fori_loop(0, NPT, _fill_const, 0)

        def _fill_ones(i, c):
            onesb[i, :] = jnp.ones((D_HID,), jnp.float32)
            return c
        lax.fori_loop(0, CH, _fill_ones, 0)

        pltpu.sync_copy(zbuf, s_sh.at[pl.ds(nbase, NPT), :])
        plsc.subcore_barrier()

        s_fire(0, 0, src=onesb)
        s_fire(1, 1, src=onesb)

        def _hist(jj, c):
            j = 2 * jj + 2
            s_wait(j - 2, 0, src=onesb)
            s_fire(j, 0, src=onesb)
            s_wait(j - 1, 1, src=onesb)
            s_fire(j + 1, 1, src=onesb)
            return c
        lax.fori_loop(0, (NCH - 2) // 2, _hist, 0)
        s_wait(NCH - 2, 0, src=onesb)
        s_wait(NCH - 1, 1, src=onesb)
        plsc.subcore_barrier()

        pltpu.sync_copy(s_sh.at[pl.ds(nbase, NPT), :], nbufS)
        pltpu.sync_copy(h_hbm.at[pl.ds(nbase, NPT), :], nbufU)

        def _setup(i, c):
            deg = nbufS[i, :] + 1.0
            r = _rsqrt16(deg)
            d2b[i, :] = 1.0 / deg
            sdb[i, :] = deg * r
            nbufU[i, :] = r * nbufU[i, :]
            return c
        lax.fori_loop(0, NPT, _setup, 0)

        pltpu.sync_copy(nbufU, u_sh.at[pl.ds(nbase, NPT), :])
        pltpu.sync_copy(zbuf, s_sh.at[pl.ds(nbase, NPT), :])
        plsc.subcore_barrier()

        taps = {6: c6_hbm, 8: c8_hbm, 10: c10_hbm}
        for p in range(1, 11):
            g_fire(0, 0)
            g_wait(0, 0)
            s_fire(0, 0)
            g_fire(1, 1)

            def _edges(jj, c):
                j1 = 2 * jj + 1
                g_wait(j1, 1)
                s_fire(j1, 1)
                s_wait(j1 - 1, 0)
                g_fire(j1 + 1, 0)
                j2 = 2 * jj + 2
                g_wait(j2, 0)
                s_fire(j2, 0)
                s_wait(j2 - 1, 1)
                g_fire(j2 + 1, 1)
                return c
            lax.fori_loop(0, (NCH - 2) // 2, _edges, 0)
            g_wait(NCH - 1, 1)
            s_fire(NCH - 1, 1)
            s_wait(NCH - 2, 0)
            s_wait(NCH - 1, 1)
            plsc.subcore_barrier()

            pltpu.sync_copy(s_sh.at[pl.ds(nbase, NPT), :], nbufS)

            def _update(i, c):
                nbufU[i, :] = d2b[i, :] * (nbufS[i, :] + nbufU[i, :])
                return c
            lax.fori_loop(0, NPT, _update, 0)

            pltpu.sync_copy(nbufU, u_sh.at[pl.ds(nbase, NPT), :])
            pltpu.sync_copy(zbuf, s_sh.at[pl.ds(nbase, NPT), :])

            if p in taps:
                def _tap(i, c):
                    nbufS[i, :] = sdb[i, :] * nbufU[i, :]
                    return c
                lax.fori_loop(0, NPT, _tap, 0)
                pltpu.sync_copy(nbufS, taps[p].at[pl.ds(nbase, NPT), :])
            plsc.subcore_barrier()


def _sc_prop(rows_r, cols_r, h):
    mesh = plsc.VectorSubcoreMesh(core_axis_name="c", subcore_axis_name="s")
    f = pl.kernel(
        _sc_body,
        out_type=(
            jax.ShapeDtypeStruct((N_NODES, D_HID), jnp.float32),
            jax.ShapeDtypeStruct((N_NODES, D_HID), jnp.float32),
            jax.ShapeDtypeStruct((N_NODES, D_HID), jnp.float32),
        ),
        mesh=mesh,
        compiler_params=pltpu.CompilerParams(use_tc_tiling_on_sc=False,
                                              needs_layout_passes=False),
        scratch_types=[
            pltpu.VMEM_SHARED((N_NODES, D_HID), jnp.float32),
            pltpu.VMEM_SHARED((N_NODES, D_HID), jnp.float32),
            pltpu.VMEM((NCH, CH), jnp.int32),
            pltpu.VMEM((NCH, CH), jnp.int32),
            pltpu.VMEM((CH, D_HID), jnp.float32),
            pltpu.VMEM((CH, D_HID), jnp.float32),
            pltpu.VMEM((CH, D_HID), jnp.float32),
            pltpu.VMEM((NPT, D_HID), jnp.float32),
            pltpu.VMEM((NPT, D_HID), jnp.float32),
            pltpu.VMEM((NPT, D_HID), jnp.float32),
            pltpu.VMEM((NPT, D_HID), jnp.float32),
            pltpu.VMEM((NPT, D_HID), jnp.float32),
            pltpu.SemaphoreType.DMA,
            pltpu.SemaphoreType.DMA,
            pltpu.SemaphoreType.DMA,
            pltpu.SemaphoreType.DMA,
        ],
    )
    return f(rows_r, cols_r, h)


D_CAT = 3 * D_HID


def _tc2_body(c6_ref, c8_ref, c10_ref, w6_ref, b6_ref, w8_ref, b8_ref,
              w10_ref, b10_ref, g2_ref, be2_ref, w2_ref, b2_ref, out_ref):
    def lin(c_ref, w_ref, b_ref):
        return lax.dot_general(c_ref[...], w_ref[...], (((1,), (1,)), ((), ())),
                               preferred_element_type=jnp.float32) + b_ref[...]

    t6 = _gelu(lin(c6_ref, w6_ref, b6_ref))
    t8 = _gelu(lin(c8_ref, w8_ref, b8_ref))
    t10 = _gelu(lin(c10_ref, w10_ref, b10_ref))

    s = jnp.sum(t6, axis=-1, keepdims=True) + jnp.sum(t8, axis=-1, keepdims=True) \
        + jnp.sum(t10, axis=-1, keepdims=True)
    mu = s / D_CAT
    v = (jnp.sum((t6 - mu) ** 2, axis=-1, keepdims=True)
         + jnp.sum((t8 - mu) ** 2, axis=-1, keepdims=True)
         + jnp.sum((t10 - mu) ** 2, axis=-1, keepdims=True)) / D_CAT
    inv = 1.0 / jnp.sqrt(v + EPS)

    g2 = g2_ref[...]
    be2 = be2_ref[...]
    w2 = w2_ref[...]
    acc = jnp.zeros_like(out_ref[...]) + b2_ref[...]
    for k, t in enumerate((t6, t8, t10)):
        nk = (t - mu) * inv * g2[:, k * D_HID:(k + 1) * D_HID] \
            + be2[:, k * D_HID:(k + 1) * D_HID]
        acc = acc + lax.dot_general(
            nk, w2[:, k * D_HID:(k + 1) * D_HID], (((1,), (1,)), ((), ())),
            preferred_element_type=jnp.float32)
    out_ref[...] = acc


def _tc2(c6, c8, c10, W6, b6, W8, b8, W10, b10, g2, be2, W2, b2):
    blk16 = pl.BlockSpec((ROWS_BLK, D_HID), lambda i: (i, 0))
    w16 = pl.BlockSpec((D_HID, D_HID), lambda i: (0, 0))
    v16 = pl.BlockSpec((1, D_HID), lambda i: (0, 0))
    v48 = pl.BlockSpec((1, D_CAT), lambda i: (0, 0))
    return pl.pallas_call(
        _tc2_body,
        out_shape=jax.ShapeDtypeStruct((N_NODES, D_OUT), jnp.float32),
        grid=(GRID_ROWS,),
        in_specs=[
            blk16, blk16, blk16,
            w16, v16, w16, v16, w16, v16,
            v48, v48,
            pl.BlockSpec((D_OUT, D_CAT), lambda i: (0, 0)),
            pl.BlockSpec((1, D_OUT), lambda i: (0, 0)),
        ],
        out_specs=pl.BlockSpec((ROWS_BLK, D_OUT), lambda i: (i, 0)),
    )(c6, c8, c10,
      W6, b6.reshape(1, D_HID), W8, b8.reshape(1, D_HID),
      W10, b10.reshape(1, D_HID),
      g2.reshape(1, D_CAT), be2.reshape(1, D_CAT),
      W2, b2.reshape(1, D_OUT))


def kernel(x, edge_index, W1, b1, W6, b6, W8, b8, W10, b10,
           g1, be1, g2, be2, W2, b2):
    h = _tc1(x, W1, b1, g1, be1)
    rows_r = edge_index[0].reshape(NCHUNKS_TOTAL, CH)
    cols_r = edge_index[1].reshape(NCHUNKS_TOTAL, CH)
    c6, c8, c10 = _sc_prop(rows_r, cols_r, h)
    return _tc2(c6, c8, c10, W6, b6, W8, b8, W10, b10, g2, be2, W2, b2)


# speedup vs baseline: 56.0347x; 1.1308x over previous
import jax
import jax.numpy as jnp
from jax import lax
from jax.experimental import pallas as pl
from jax.experimental.pallas import tpu as pltpu
from jax.experimental.pallas import tpu_sc as plsc

N_NODES = 10000
N_EDGES = 320000
D_IN = 128
D_HID = 16
D_OUT = 128

NTILES = 16
NPT = N_NODES // NTILES
CH = 125
NCHUNKS_TOTAL = N_EDGES // CH
NCH = NCHUNKS_TOTAL // NTILES
EPS = 1e-5


ROWS_BLK = 1000
GRID_ROWS = N_NODES // ROWS_BLK


def _gelu(x):
    return 0.5 * x * (1.0 + lax.erf(x * (2.0 ** -0.5)))


def _tc1_body(x_ref, w1_ref, b1_ref, g1_ref, be1_ref, h_ref):
    x = x_ref[...]
    h = lax.dot_general(x, w1_ref[...], (((1,), (1,)), ((), ())),
                        preferred_element_type=jnp.float32)
    h = _gelu(h + b1_ref[...])
    mu = jnp.mean(h, axis=-1, keepdims=True)
    var = jnp.mean((h - mu) ** 2, axis=-1, keepdims=True)
    h_ref[...] = (h - mu) / jnp.sqrt(var + EPS) * g1_ref[...] + be1_ref[...]


def _tc1(x, W1, b1, g1, be1):
    return pl.pallas_call(
        _tc1_body,
        out_shape=jax.ShapeDtypeStruct((N_NODES, D_HID), jnp.float32),
        grid=(GRID_ROWS,),
        in_specs=[
            pl.BlockSpec((ROWS_BLK, D_IN), lambda i: (i, 0)),
            pl.BlockSpec((D_HID, D_IN), lambda i: (0, 0)),
            pl.BlockSpec((1, D_HID), lambda i: (0, 0)),
            pl.BlockSpec((1, D_HID), lambda i: (0, 0)),
            pl.BlockSpec((1, D_HID), lambda i: (0, 0)),
        ],
        out_specs=pl.BlockSpec((ROWS_BLK, D_HID), lambda i: (i, 0)),
    )(x, W1, b1.reshape(1, D_HID), g1.reshape(1, D_HID), be1.reshape(1, D_HID))


def _rsqrt16(x):
    i = plsc.bitcast(x, jnp.int32)
    i = jnp.int32(0x5F3759DF) - (i >> 1)
    y = plsc.bitcast(i, jnp.float32)
    for _ in range(3):
        y = y * (1.5 - 0.5 * x * y * y)
    return y


def _sc_body(rows_hbm, cols_hbm, h_hbm, c6_hbm, c8_hbm, c10_hbm,
             u_sh, s_sh, rowix, colix, gbuf0, gbuf1, gbuf2, gbuf3, onesb,
             nbufS, nbufU, d2b, sdb, zbuf,
             gsem0, gsem1, gsem2, gsem3, ssem0, ssem1, ssem2, ssem3):
    cid = lax.axis_index("c")
    tid = lax.axis_index("s")

    @pl.when(cid == 0)
    def _work():
        nbase = tid * NPT
        cbase = tid * NCH

        pltpu.sync_copy(rows_hbm.at[pl.ds(cbase, NCH), :], rowix)
        pltpu.sync_copy(cols_hbm.at[pl.ds(cbase, NCH), :], colix)

        gbufs = (gbuf0, gbuf1, gbuf2, gbuf3)
        gsems = (gsem0, gsem1, gsem2, gsem3)
        ssems = (ssem0, ssem1, ssem2, ssem3)

        def g_fire(j, b):
            pltpu.async_copy(u_sh.at[rowix.at[j]], gbufs[b], gsems[b])

        def g_wait(j, b):
            pltpu.make_async_copy(u_sh.at[rowix.at[j]], gbufs[b],
                                  gsems[b]).wait()

        def s_fire(j, b, src=None):
            pltpu.async_copy(src if src is not None else gbufs[b],
                             s_sh.at[colix.at[j]], ssems[b], add=True)

        def s_wait(j, b, src=None):
            pltpu.make_async_copy(src if src is not None else gbufs[b],
                                  s_sh.at[colix.at[j]], ssems[b]).wait()

        def _fill_const(i, c):
            zbuf[i, :] = jnp.zeros((D_HID,), jnp.float32)
            return c
        lax.fori_loop(0, NPT, _fill_const, 0)

        def _fill_ones(i, c):
            onesb[i, :] = jnp.ones((D_HID,), jnp.float32)
            return c
        lax.fori_loop(0, CH, _fill_ones, 0)

        pltpu.sync_copy(zbuf, s_sh.at[pl.ds(nbase, NPT), :])
        plsc.subcore_barrier()

        for b in range(4):
            s_fire(b, b, src=onesb)

        def _hist(jj, c):
            for b in range(4):
                j = 4 * jj + 4 + b
                s_wait(j - 4, b, src=onesb)
                s_fire(j, b, src=onesb)
            return c
        lax.fori_loop(0, (NCH - 4) // 4, _hist, 0)
        for b in range(4):
            s_wait(NCH - 4 + b, b, src=onesb)
        plsc.subcore_barrier()

        pltpu.sync_copy(s_sh.at[pl.ds(nbase, NPT), :], nbufS)
        pltpu.sync_copy(h_hbm.at[pl.ds(nbase, NPT), :], nbufU)

        def _setup(i, c):
            deg = nbufS[i, :] + 1.0
            r = _rsqrt16(deg)
            d2b[i, :] = 1.0 / deg
            sdb[i, :] = deg * r
            nbufU[i, :] = r * nbufU[i, :]
            return c
        lax.fori_loop(0, NPT, _setup, 0)

        pltpu.sync_copy(nbufU, u_sh.at[pl.ds(nbase, NPT), :])
        pltpu.sync_copy(zbuf, s_sh.at[pl.ds(nbase, NPT), :])
        plsc.subcore_barrier()

        taps = {6: c6_hbm, 8: c8_hbm, 10: c10_hbm}
        for p in range(1, 11):
            g_fire(0, 0)
            g_fire(1, 1)
            g_wait(0, 0)
            s_fire(0, 0)
            g_fire(2, 2)
            g_wait(1, 1)
            s_fire(1, 1)
            g_fire(3, 3)

            def _edges(jj, c):
                for k in range(4):
                    j = 4 * jj + 2 + k
                    b = (2 + k) % 4
                    g_wait(j, b)
                    s_fire(j, b)
                    s_wait(j - 2, (b + 2) % 4)
                    g_fire(j + 2, (b + 2) % 4)
                return c
            lax.fori_loop(0, (NCH - 4) // 4, _edges, 0)
            g_wait(NCH - 2, 2)
            s_fire(NCH - 2, 2)
            s_wait(NCH - 4, 0)
            g_wait(NCH - 1, 3)
            s_fire(NCH - 1, 3)
            s_wait(NCH - 3, 1)
            s_wait(NCH - 2, 2)
            s_wait(NCH - 1, 3)
            plsc.subcore_barrier()

            pltpu.sync_copy(s_sh.at[pl.ds(nbase, NPT), :], nbufS)

            def _update(i, c):
                nbufU[i, :] = d2b[i, :] * (nbufS[i, :] + nbufU[i, :])
                return c
            lax.fori_loop(0, NPT, _update, 0)

            pltpu.sync_copy(nbufU, u_sh.at[pl.ds(nbase, NPT), :])
            pltpu.sync_copy(zbuf, s_sh.at[pl.ds(nbase, NPT), :])

            if p in taps:
                def _tap(i, c):
                    nbufS[i, :] = sdb[i, :] * nbufU[i, :]
                    return c
                lax.fori_loop(0, NPT, _tap, 0)
                pltpu.sync_copy(nbufS, taps[p].at[pl.ds(nbase, NPT), :])
            plsc.subcore_barrier()


def _sc_prop(rows_r, cols_r, h):
    mesh = plsc.VectorSubcoreMesh(core_axis_name="c", subcore_axis_name="s")
    f = pl.kernel(
        _sc_body,
        out_type=(
            jax.ShapeDtypeStruct((N_NODES, D_HID), jnp.float32),
            jax.ShapeDtypeStruct((N_NODES, D_HID), jnp.float32),
            jax.ShapeDtypeStruct((N_NODES, D_HID), jnp.float32),
        ),
        mesh=mesh,
        compiler_params=pltpu.CompilerParams(use_tc_tiling_on_sc=False,
                                              needs_layout_passes=False),
        scratch_types=[
            pltpu.VMEM_SHARED((N_NODES, D_HID), jnp.float32),
            pltpu.VMEM_SHARED((N_NODES, D_HID), jnp.float32),
            pltpu.VMEM((NCH, CH), jnp.int32),
            pltpu.VMEM((NCH, CH), jnp.int32),
            pltpu.VMEM((CH, D_HID), jnp.float32),
            pltpu.VMEM((CH, D_HID), jnp.float32),
            pltpu.VMEM((CH, D_HID), jnp.float32),
            pltpu.VMEM((CH, D_HID), jnp.float32),
            pltpu.VMEM((CH, D_HID), jnp.float32),
            pltpu.VMEM((NPT, D_HID), jnp.float32),
            pltpu.VMEM((NPT, D_HID), jnp.float32),
            pltpu.VMEM((NPT, D_HID), jnp.float32),
            pltpu.VMEM((NPT, D_HID), jnp.float32),
            pltpu.VMEM((NPT, D_HID), jnp.float32),
        ] + [pltpu.SemaphoreType.DMA] * 8,
    )
    return f(rows_r, cols_r, h)


D_CAT = 3 * D_HID


def _tc2_body(c6_ref, c8_ref, c10_ref, w6_ref, b6_ref, w8_ref, b8_ref,
              w10_ref, b10_ref, g2_ref, be2_ref, w2_ref, b2_ref, out_ref):
    def lin(c_ref, w_ref, b_ref):
        return lax.dot_general(c_ref[...], w_ref[...], (((1,), (1,)), ((), ())),
                               preferred_element_type=jnp.float32) + b_ref[...]

    t6 = _gelu(lin(c6_ref, w6_ref, b6_ref))
    t8 = _gelu(lin(c8_ref, w8_ref, b8_ref))
    t10 = _gelu(lin(c10_ref, w10_ref, b10_ref))

    s = jnp.sum(t6, axis=-1, keepdims=True) + jnp.sum(t8, axis=-1, keepdims=True) \
        + jnp.sum(t10, axis=-1, keepdims=True)
    mu = s / D_CAT
    v = (jnp.sum((t6 - mu) ** 2, axis=-1, keepdims=True)
         + jnp.sum((t8 - mu) ** 2, axis=-1, keepdims=True)
         + jnp.sum((t10 - mu) ** 2, axis=-1, keepdims=True)) / D_CAT
    inv = 1.0 / jnp.sqrt(v + EPS)

    g2 = g2_ref[...]
    be2 = be2_ref[...]
    w2 = w2_ref[...]
    acc = jnp.zeros_like(out_ref[...]) + b2_ref[...]
    for k, t in enumerate((t6, t8, t10)):
        nk = (t - mu) * inv * g2[:, k * D_HID:(k + 1) * D_HID] \
            + be2[:, k * D_HID:(k + 1) * D_HID]
        acc = acc + lax.dot_general(
            nk, w2[:, k * D_HID:(k + 1) * D_HID], (((1,), (1,)), ((), ())),
            preferred_element_type=jnp.float32)
    out_ref[...] = acc


def _tc2(c6, c8, c10, W6, b6, W8, b8, W10, b10, g2, be2, W2, b2):
    blk16 = pl.BlockSpec((ROWS_BLK, D_HID), lambda i: (i, 0))
    w16 = pl.BlockSpec((D_HID, D_HID), lambda i: (0, 0))
    v16 = pl.BlockSpec((1, D_HID), lambda i: (0, 0))
    v48 = pl.BlockSpec((1, D_CAT), lambda i: (0, 0))
    return pl.pallas_call(
        _tc2_body,
        out_shape=jax.ShapeDtypeStruct((N_NODES, D_OUT), jnp.float32),
        grid=(GRID_ROWS,),
        in_specs=[
            blk16, blk16, blk16,
            w16, v16, w16, v16, w16, v16,
            v48, v48,
            pl.BlockSpec((D_OUT, D_CAT), lambda i: (0, 0)),
            pl.BlockSpec((1, D_OUT), lambda i: (0, 0)),
        ],
        out_specs=pl.BlockSpec((ROWS_BLK, D_OUT), lambda i: (i, 0)),
    )(c6, c8, c10,
      W6, b6.reshape(1, D_HID), W8, b8.reshape(1, D_HID),
      W10, b10.reshape(1, D_HID),
      g2.reshape(1, D_CAT), be2.reshape(1, D_CAT),
      W2, b2.reshape(1, D_OUT))


def kernel(x, edge_index, W1, b1, W6, b6, W8, b8, W10, b10,
           g1, be1, g2, be2, W2, b2):
    h = _tc1(x, W1, b1, g1, be1)
    rows_r = edge_index[0].reshape(NCHUNKS_TOTAL, CH)
    cols_r = edge_index[1].reshape(NCHUNKS_TOTAL, CH)
    c6, c8, c10 = _sc_prop(rows_r, cols_r, h)
    return _tc2(c6, c8, c10, W6, b6, W8, b8, W10, b10, g2, be2, W2, b2)


# speedup vs baseline: 56.0364x; 1.0000x over previous
import jax
import jax.numpy as jnp
from jax import lax
from jax.experimental import pallas as pl
from jax.experimental.pallas import tpu as pltpu
from jax.experimental.pallas import tpu_sc as plsc

N_NODES = 10000
N_EDGES = 320000
D_IN = 128
D_HID = 16
D_OUT = 128

NTILES = 16
NPT = N_NODES // NTILES
CH = 125
NCHUNKS_TOTAL = N_EDGES // CH
NCH = NCHUNKS_TOTAL // NTILES
EPS = 1e-5


ROWS_BLK = 1000
GRID_ROWS = N_NODES // ROWS_BLK


def _gelu(x):
    return 0.5 * x * (1.0 + lax.erf(x * (2.0 ** -0.5)))


def _tc1_body(x_ref, w1_ref, b1_ref, g1_ref, be1_ref, h_ref):
    x = x_ref[...]
    h = lax.dot_general(x, w1_ref[...], (((1,), (1,)), ((), ())),
                        preferred_element_type=jnp.float32)
    h = _gelu(h + b1_ref[...])
    mu = jnp.mean(h, axis=-1, keepdims=True)
    var = jnp.mean((h - mu) ** 2, axis=-1, keepdims=True)
    h_ref[...] = (h - mu) / jnp.sqrt(var + EPS) * g1_ref[...] + be1_ref[...]


def _tc1(x, W1, b1, g1, be1):
    return pl.pallas_call(
        _tc1_body,
        out_shape=jax.ShapeDtypeStruct((N_NODES, D_HID), jnp.float32),
        grid=(GRID_ROWS,),
        in_specs=[
            pl.BlockSpec((ROWS_BLK, D_IN), lambda i: (i, 0)),
            pl.BlockSpec((D_HID, D_IN), lambda i: (0, 0)),
            pl.BlockSpec((1, D_HID), lambda i: (0, 0)),
            pl.BlockSpec((1, D_HID), lambda i: (0, 0)),
            pl.BlockSpec((1, D_HID), lambda i: (0, 0)),
        ],
        out_specs=pl.BlockSpec((ROWS_BLK, D_HID), lambda i: (i, 0)),
    )(x, W1, b1.reshape(1, D_HID), g1.reshape(1, D_HID), be1.reshape(1, D_HID))


def _rsqrt16(x):
    i = plsc.bitcast(x, jnp.int32)
    i = jnp.int32(0x5F3759DF) - (i >> 1)
    y = plsc.bitcast(i, jnp.float32)
    for _ in range(3):
        y = y * (1.5 - 0.5 * x * y * y)
    return y


def _sc_body(rows_hbm, cols_hbm, h_hbm, c6_hbm, c8_hbm, c10_hbm,
             u_sh, s_sh, rowix, colix, gbuf0, gbuf1, gbuf2, gbuf3, onesb,
             nbufS, nbufU, d2b, sdb, zbuf,
             gsem0, gsem1, gsem2, gsem3, ssem0, ssem1, ssem2, ssem3):
    cid = lax.axis_index("c")
    tid = lax.axis_index("s")

    @pl.when(cid == 0)
    def _work():
        nbase = tid * NPT
        cbase = tid * NCH

        pltpu.sync_copy(rows_hbm.at[pl.ds(cbase, NCH), :], rowix)
        pltpu.sync_copy(cols_hbm.at[pl.ds(cbase, NCH), :], colix)

        gbufs = (gbuf0, gbuf1, gbuf2, gbuf3)
        gsems = (gsem0, gsem1, gsem2, gsem3)
        ssems = (ssem0, ssem1, ssem2, ssem3)

        def g_fire(j, b):
            pltpu.async_copy(u_sh.at[rowix.at[j]], gbufs[b], gsems[b])

        def g_wait(j, b):
            pltpu.make_async_copy(u_sh.at[rowix.at[j]], gbufs[b],
                                  gsems[b]).wait()

        def s_fire(j, b, src=None):
            pltpu.async_copy(src if src is not None else gbufs[b],
                             s_sh.at[colix.at[j]], ssems[b], add=True)

        def s_wait(j, b, src=None):
            pltpu.make_async_copy(src if src is not None else gbufs[b],
                                  s_sh.at[colix.at[j]], ssems[b]).wait()

        def _fill_const(i, c):
            zbuf[i, :] = jnp.zeros((D_HID,), jnp.float32)
            return c
        lax.fori_loop(0, NPT, _fill_const, 0)

        def _fill_ones(i, c):
            onesb[i, :] = jnp.ones((D_HID,), jnp.float32)
            return c
        lax.fori_loop(0, CH, _fill_ones, 0)

        pltpu.sync_copy(zbuf, s_sh.at[pl.ds(nbase, NPT), :])
        plsc.subcore_barrier()

        for b in range(4):
            s_fire(b, b, src=onesb)

        def _hist(jj, c):
            for b in range(4):
                j = 4 * jj + 4 + b
                s_wait(j - 4, b, src=onesb)
                s_fire(j, b, src=onesb)
            return c
        lax.fori_loop(0, (NCH - 4) // 4, _hist, 0)
        for b in range(4):
            s_wait(NCH - 4 + b, b, src=onesb)
        plsc.subcore_barrier()

        pltpu.sync_copy(s_sh.at[pl.ds(nbase, NPT), :], nbufS)
        pltpu.sync_copy(h_hbm.at[pl.ds(nbase, NPT), :], nbufU)

        def _setup(i, c):
            deg = nbufS[i, :] + 1.0
            r = _rsqrt16(deg)
            d2b[i, :] = 1.0 / deg
            sdb[i, :] = deg * r
            nbufU[i, :] = r * nbufU[i, :]
            return c
        lax.fori_loop(0, NPT, _setup, 0)

        pltpu.sync_copy(nbufU, u_sh.at[pl.ds(nbase, NPT), :])
        pltpu.sync_copy(zbuf, s_sh.at[pl.ds(nbase, NPT), :])
        plsc.subcore_barrier()

        taps = {6: c6_hbm, 8: c8_hbm, 10: c10_hbm}
        for p in range(1, 11):
          with jax.named_scope(f"edge_phase_{p}"):
            g_fire(0, 0)
            g_fire(1, 1)
            g_wait(0, 0)
            s_fire(0, 0)
            g_fire(2, 2)
            g_wait(1, 1)
            s_fire(1, 1)
            g_fire(3, 3)

            def _edges(jj, c):
                for k in range(4):
                    j = 4 * jj + 2 + k
                    b = (2 + k) % 4
                    g_wait(j, b)
                    s_fire(j, b)
                    s_wait(j - 2, (b + 2) % 4)
                    g_fire(j + 2, (b + 2) % 4)
                return c
            lax.fori_loop(0, (NCH - 4) // 4, _edges, 0)
            g_wait(NCH - 2, 2)
            s_fire(NCH - 2, 2)
            s_wait(NCH - 4, 0)
            g_wait(NCH - 1, 3)
            s_fire(NCH - 1, 3)
            s_wait(NCH - 3, 1)
            s_wait(NCH - 2, 2)
            s_wait(NCH - 1, 3)
            plsc.subcore_barrier()

          with jax.named_scope(f"node_phase_{p}"):
            pltpu.sync_copy(s_sh.at[pl.ds(nbase, NPT), :], nbufS)

            def _update(i, c):
                nbufU[i, :] = d2b[i, :] * (nbufS[i, :] + nbufU[i, :])
                return c
            lax.fori_loop(0, NPT, _update, 0)

            pltpu.sync_copy(nbufU, u_sh.at[pl.ds(nbase, NPT), :])
            pltpu.sync_copy(zbuf, s_sh.at[pl.ds(nbase, NPT), :])

            if p in taps:
                def _tap(i, c):
                    nbufS[i, :] = sdb[i, :] * nbufU[i, :]
                    return c
                lax.fori_loop(0, NPT, _tap, 0)
                pltpu.sync_copy(nbufS, taps[p].at[pl.ds(nbase, NPT), :])
            plsc.subcore_barrier()


def _sc_prop(rows_r, cols_r, h):
    mesh = plsc.VectorSubcoreMesh(core_axis_name="c", subcore_axis_name="s")
    f = pl.kernel(
        _sc_body,
        out_type=(
            jax.ShapeDtypeStruct((N_NODES, D_HID), jnp.float32),
            jax.ShapeDtypeStruct((N_NODES, D_HID), jnp.float32),
            jax.ShapeDtypeStruct((N_NODES, D_HID), jnp.float32),
        ),
        mesh=mesh,
        compiler_params=pltpu.CompilerParams(use_tc_tiling_on_sc=False,
                                              needs_layout_passes=False),
        scratch_types=[
            pltpu.VMEM_SHARED((N_NODES, D_HID), jnp.float32),
            pltpu.VMEM_SHARED((N_NODES, D_HID), jnp.float32),
            pltpu.VMEM((NCH, CH), jnp.int32),
            pltpu.VMEM((NCH, CH), jnp.int32),
            pltpu.VMEM((CH, D_HID), jnp.float32),
            pltpu.VMEM((CH, D_HID), jnp.float32),
            pltpu.VMEM((CH, D_HID), jnp.float32),
            pltpu.VMEM((CH, D_HID), jnp.float32),
            pltpu.VMEM((CH, D_HID), jnp.float32),
            pltpu.VMEM((NPT, D_HID), jnp.float32),
            pltpu.VMEM((NPT, D_HID), jnp.float32),
            pltpu.VMEM((NPT, D_HID), jnp.float32),
            pltpu.VMEM((NPT, D_HID), jnp.float32),
            pltpu.VMEM((NPT, D_HID), jnp.float32),
        ] + [pltpu.SemaphoreType.DMA] * 8,
    )
    return f(rows_r, cols_r, h)


D_CAT = 3 * D_HID


def _tc2_body(c6_ref, c8_ref, c10_ref, w6_ref, b6_ref, w8_ref, b8_ref,
              w10_ref, b10_ref, g2_ref, be2_ref, w2_ref, b2_ref, out_ref):
    def lin(c_ref, w_ref, b_ref):
        return lax.dot_general(c_ref[...], w_ref[...], (((1,), (1,)), ((), ())),
                               preferred_element_type=jnp.float32) + b_ref[...]

    t6 = _gelu(lin(c6_ref, w6_ref, b6_ref))
    t8 = _gelu(lin(c8_ref, w8_ref, b8_ref))
    t10 = _gelu(lin(c10_ref, w10_ref, b10_ref))

    s = jnp.sum(t6, axis=-1, keepdims=True) + jnp.sum(t8, axis=-1, keepdims=True) \
        + jnp.sum(t10, axis=-1, keepdims=True)
    mu = s / D_CAT
    v = (jnp.sum((t6 - mu) ** 2, axis=-1, keepdims=True)
         + jnp.sum((t8 - mu) ** 2, axis=-1, keepdims=True)
         + jnp.sum((t10 - mu) ** 2, axis=-1, keepdims=True)) / D_CAT
    inv = 1.0 / jnp.sqrt(v + EPS)

    g2 = g2_ref[...]
    be2 = be2_ref[...]
    w2 = w2_ref[...]
    acc = jnp.zeros_like(out_ref[...]) + b2_ref[...]
    for k, t in enumerate((t6, t8, t10)):
        nk = (t - mu) * inv * g2[:, k * D_HID:(k + 1) * D_HID] \
            + be2[:, k * D_HID:(k + 1) * D_HID]
        acc = acc + lax.dot_general(
            nk, w2[:, k * D_HID:(k + 1) * D_HID], (((1,), (1,)), ((), ())),
            preferred_element_type=jnp.float32)
    out_ref[...] = acc


def _tc2(c6, c8, c10, W6, b6, W8, b8, W10, b10, g2, be2, W2, b2):
    blk16 = pl.BlockSpec((ROWS_BLK, D_HID), lambda i: (i, 0))
    w16 = pl.BlockSpec((D_HID, D_HID), lambda i: (0, 0))
    v16 = pl.BlockSpec((1, D_HID), lambda i: (0, 0))
    v48 = pl.BlockSpec((1, D_CAT), lambda i: (0, 0))
    return pl.pallas_call(
        _tc2_body,
        out_shape=jax.ShapeDtypeStruct((N_NODES, D_OUT), jnp.float32),
        grid=(GRID_ROWS,),
        in_specs=[
            blk16, blk16, blk16,
            w16, v16, w16, v16, w16, v16,
            v48, v48,
            pl.BlockSpec((D_OUT, D_CAT), lambda i: (0, 0)),
            pl.BlockSpec((1, D_OUT), lambda i: (0, 0)),
        ],
        out_specs=pl.BlockSpec((ROWS_BLK, D_OUT), lambda i: (i, 0)),
    )(c6, c8, c10,
      W6, b6.reshape(1, D_HID), W8, b8.reshape(1, D_HID),
      W10, b10.reshape(1, D_HID),
      g2.reshape(1, D_CAT), be2.reshape(1, D_CAT),
      W2, b2.reshape(1, D_OUT))


def kernel(x, edge_index, W1, b1, W6, b6, W8, b8, W10, b10,
           g1, be1, g2, be2, W2, b2):
    h = _tc1(x, W1, b1, g1, be1)
    rows_r = edge_index[0].reshape(NCHUNKS_TOTAL, CH)
    cols_r = edge_index[1].reshape(NCHUNKS_TOTAL, CH)
    c6, c8, c10 = _sc_prop(rows_r, cols_r, h)
    return _tc2(c6, c8, c10, W6, b6, W8, b8, W10, b10, g2, be2, W2, b2)
